# Initial kernel scaffold; baseline (speedup 1.0000x reference)
import jax
import jax.numpy as jnp
from jax.experimental import pallas as pl


def kernel(x, table, W1, b1, W2, b2):
    raise NotImplementedError("write your pallas kernel here")


# speedup vs baseline: 12.6175x; 12.6175x over previous
import functools

import jax
import jax.numpy as jnp
from jax import lax
from jax.experimental import pallas as pl
from jax.experimental.pallas import tpu as pltpu
from jax.experimental.pallas import tpu_sc as plsc

NC = 2
NS = 16
NW = NC * NS


def _make_pool(B, L, E, CB):
    RW = B // NW
    NCHUNK = RW // CB
    mesh = plsc.VectorSubcoreMesh(
        core_axis_name="c", subcore_axis_name="s",
        num_cores=NC, num_subcores=NS)

    @functools.partial(
        pl.kernel,
        mesh=mesh,
        compiler_params=pltpu.CompilerParams(use_tc_tiling_on_sc=False),
        out_type=jax.ShapeDtypeStruct((B * E,), jnp.float32),
        scratch_types=[
            pltpu.VMEM((CB * L,), jnp.int32),
            pltpu.VMEM((CB * L, E), jnp.float32),
            pltpu.VMEM((RW * E,), jnp.float32),
            pltpu.SemaphoreType.DMA,
        ],
    )
    def pool(x_hbm, table_hbm, out_hbm, idx_v, rows_v, out_v, sem):
        wid = lax.axis_index("s") * NC + lax.axis_index("c")
        base = wid * RW

        def chunk(g, carry):
            off = (base + g * CB) * L
            pltpu.sync_copy(x_hbm.at[pl.ds(off, CB * L)], idx_v)
            pltpu.async_copy(table_hbm.at[idx_v], rows_v, sem).wait()
            for b in range(CB):
                r0 = b * L

                def rbody(i, acc):
                    a0, a1, a2, a3 = acc
                    r = r0 + 2 * i
                    a0 = a0 + rows_v[r, pl.ds(0, 16)]
                    a1 = a1 + rows_v[r, pl.ds(16, 16)]
                    a2 = a2 + rows_v[r + 1, pl.ds(0, 16)]
                    a3 = a3 + rows_v[r + 1, pl.ds(16, 16)]
                    return (a0, a1, a2, a3)

                z = jnp.zeros((16,), jnp.float32)
                a0, a1, a2, a3 = lax.fori_loop(0, L // 2, rbody, (z, z, z, z))
                ob = (g * CB + b) * E
                out_v[pl.ds(ob, 16)] = a0 + a2
                out_v[pl.ds(ob + 16, 16)] = a1 + a3
            return carry

        lax.fori_loop(0, NCHUNK, chunk, 0)
        pltpu.sync_copy(out_v, out_hbm.at[pl.ds(base * E, RW * E)])

    return pool


def _mlp_body(h_ref, w1_ref, b1_ref, w2_ref, b2_ref, o_ref):
    h = h_ref[...]
    z = jnp.dot(h, w1_ref[...], preferred_element_type=jnp.float32) + b1_ref[...]
    z = jnp.maximum(z, 0.0)
    o = jnp.dot(z, w2_ref[...], preferred_element_type=jnp.float32) + b2_ref[...]
    o_ref[...] = jax.nn.sigmoid(o)


def _make_mlp(B, E, H, O, BB):
    grid = (B // BB,)
    return pl.pallas_call(
        _mlp_body,
        grid=grid,
        in_specs=[
            pl.BlockSpec((BB, E), lambda i: (i, 0)),
            pl.BlockSpec((E, H), lambda i: (0, 0)),
            pl.BlockSpec((1, H), lambda i: (0, 0)),
            pl.BlockSpec((H, O), lambda i: (0, 0)),
            pl.BlockSpec((1, O), lambda i: (0, 0)),
        ],
        out_specs=pl.BlockSpec((BB, O), lambda i: (i, 0)),
        out_shape=jax.ShapeDtypeStruct((B, O), jnp.float32),
    )


def kernel(x, table, W1, b1, W2, b2):
    B, L = x.shape
    E = table.shape[1]
    H = W1.shape[1]
    O = W2.shape[1]
    xf = x.reshape(-1).astype(jnp.int32)
    pool = _make_pool(B, L, E, CB=8)
    hsum = pool(xf, table)
    h = hsum.reshape(B, E)
    mlp = _make_mlp(B, E, H, O, BB=2048)
    return mlp(h, W1 * (1.0 / L), b1.reshape(1, H), W2, b2.reshape(1, O))


# speedup vs baseline: 16.7183x; 1.3250x over previous
import functools

import jax
import jax.numpy as jnp
from jax import lax
from jax.experimental import pallas as pl
from jax.experimental.pallas import tpu as pltpu
from jax.experimental.pallas import tpu_sc as plsc

NC = 2
NS = 16
NW = NC * NS


def _make_pool(B, L, E, CB):
    RW = B // NW
    NCHUNK = RW // CB
    NPAIR = NCHUNK // 2
    mesh = plsc.VectorSubcoreMesh(
        core_axis_name="c", subcore_axis_name="s",
        num_cores=NC, num_subcores=NS)

    @functools.partial(
        pl.kernel,
        mesh=mesh,
        compiler_params=pltpu.CompilerParams(use_tc_tiling_on_sc=False),
        out_type=jax.ShapeDtypeStruct((B * E,), jnp.float32),
        scratch_types=[
            pltpu.VMEM((CB * L,), jnp.int32),
            pltpu.VMEM((CB * L,), jnp.int32),
            pltpu.VMEM((CB * L, E), jnp.float32),
            pltpu.VMEM((CB * L, E), jnp.float32),
            pltpu.VMEM((RW * E,), jnp.float32),
            pltpu.SemaphoreType.DMA,
            pltpu.SemaphoreType.DMA,
            pltpu.SemaphoreType.DMA,
        ],
    )
    def pool(x_hbm, table_hbm, out_hbm,
             idx_a, idx_b, rows_a, rows_b, out_v, sem_a, sem_b, sem_i):
        wid = lax.axis_index("s") * NC + lax.axis_index("c")
        row0 = wid * RW

        def stage(g, idx_v):
            r = row0 + g * CB
            for b in range(CB):
                pltpu.async_copy(x_hbm.at[r + b], idx_v.at[pl.ds(b * L, L)],
                                 sem_i)

        def drain_stage(g, idx_v):
            r = row0 + g * CB
            for b in range(CB):
                pltpu.make_async_copy(x_hbm.at[r + b],
                                      idx_v.at[pl.ds(b * L, L)],
                                      sem_i).wait()

        def reduce(rows_v, g):
            for b in range(CB):
                r0 = b * L

                def rbody(i, acc):
                    a0, a1, a2, a3, a4, a5, a6, a7 = acc
                    r = r0 + 4 * i
                    a0 = a0 + rows_v[r, pl.ds(0, 16)]
                    a1 = a1 + rows_v[r, pl.ds(16, 16)]
                    a2 = a2 + rows_v[r + 1, pl.ds(0, 16)]
                    a3 = a3 + rows_v[r + 1, pl.ds(16, 16)]
                    a4 = a4 + rows_v[r + 2, pl.ds(0, 16)]
                    a5 = a5 + rows_v[r + 2, pl.ds(16, 16)]
                    a6 = a6 + rows_v[r + 3, pl.ds(0, 16)]
                    a7 = a7 + rows_v[r + 3, pl.ds(16, 16)]
                    return (a0, a1, a2, a3, a4, a5, a6, a7)

                z = jnp.zeros((16,), jnp.float32)
                a0, a1, a2, a3, a4, a5, a6, a7 = lax.fori_loop(
                    0, L // 4, rbody, (z, z, z, z, z, z, z, z))
                ob = (g * CB + b) * E
                out_v[pl.ds(ob, 16)] = (a0 + a2) + (a4 + a6)
                out_v[pl.ds(ob + 16, 16)] = (a1 + a3) + (a5 + a7)

        stage(0, idx_a)
        drain_stage(0, idx_a)
        pltpu.async_copy(table_hbm.at[idx_a], rows_a, sem_a)
        stage(1, idx_b)
        drain_stage(1, idx_b)
        pltpu.async_copy(table_hbm.at[idx_b], rows_b, sem_b)

        def pair(p, carry):
            g0 = 2 * p
            not_last = p + 1 < NPAIR

            pltpu.make_async_copy(table_hbm.at[idx_a], rows_a, sem_a).wait()

            @pl.when(not_last)
            def _():
                stage(g0 + 2, idx_a)

            reduce(rows_a, g0)

            @pl.when(not_last)
            def _():
                drain_stage(g0 + 2, idx_a)
                pltpu.async_copy(table_hbm.at[idx_a], rows_a, sem_a)

            pltpu.make_async_copy(table_hbm.at[idx_b], rows_b, sem_b).wait()

            @pl.when(not_last)
            def _():
                stage(g0 + 3, idx_b)

            reduce(rows_b, g0 + 1)

            @pl.when(not_last)
            def _():
                drain_stage(g0 + 3, idx_b)
                pltpu.async_copy(table_hbm.at[idx_b], rows_b, sem_b)

            return carry

        lax.fori_loop(0, NPAIR, pair, 0)
        pltpu.sync_copy(out_v, out_hbm.at[pl.ds(row0 * E, RW * E)])

    return pool


def _mlp_body(h_ref, w1_ref, b1_ref, w2_ref, b2_ref, o_ref):
    h = h_ref[...]
    z = jnp.dot(h, w1_ref[...], preferred_element_type=jnp.float32) + b1_ref[...]
    z = jnp.maximum(z, 0.0)
    o = jnp.dot(z, w2_ref[...], preferred_element_type=jnp.float32) + b2_ref[...]
    o_ref[...] = jax.nn.sigmoid(o)


def _make_mlp(B, E, H, O, BB):
    grid = (B // BB,)
    return pl.pallas_call(
        _mlp_body,
        grid=grid,
        in_specs=[
            pl.BlockSpec((BB, E), lambda i: (i, 0)),
            pl.BlockSpec((E, H), lambda i: (0, 0)),
            pl.BlockSpec((1, H), lambda i: (0, 0)),
            pl.BlockSpec((H, O), lambda i: (0, 0)),
            pl.BlockSpec((1, O), lambda i: (0, 0)),
        ],
        out_specs=pl.BlockSpec((BB, O), lambda i: (i, 0)),
        out_shape=jax.ShapeDtypeStruct((B, O), jnp.float32),
    )


def kernel(x, table, W1, b1, W2, b2):
    B, L = x.shape
    E = table.shape[1]
    H = W1.shape[1]
    O = W2.shape[1]
    pool = _make_pool(B, L, E, CB=8)
    hsum = pool(x.astype(jnp.int32), table)
    h = hsum.reshape(B, E)
    mlp = _make_mlp(B, E, H, O, BB=2048)
    return mlp(h, W1 * (1.0 / L), b1.reshape(1, H), W2, b2.reshape(1, O))
